# baseline (speedup 1.0000x reference)
import functools

import jax
import jax.numpy as jnp
from jax import lax
from jax.experimental import pallas as pl
from jax.experimental.pallas import tpu as pltpu
from jax.experimental.pallas import tpu_sc as plsc

B = 16384
D = 64
H = 256
NC, NS = 2, 16
NW = NC * NS
BPW = B // NW

@functools.lru_cache(maxsize=None)
def _gather_kernel():
    mesh = plsc.VectorSubcoreMesh(
        core_axis_name="c", subcore_axis_name="s", num_cores=NC, num_subcores=NS
    )

    @functools.partial(
        pl.kernel,
        out_type=(
            jax.ShapeDtypeStruct((B, D), jnp.float32),
            jax.ShapeDtypeStruct((B, D), jnp.float32),
        ),
        mesh=mesh,
        compiler_params=pltpu.CompilerParams(use_tc_tiling_on_sc=False),
        scratch_types=[
            pltpu.VMEM((BPW,), jnp.int32),
            pltpu.VMEM((BPW,), jnp.int32),
            pltpu.VMEM((BPW, D), jnp.float32),
            pltpu.VMEM((BPW, D), jnp.float32),
            pltpu.SemaphoreType.DMA,
            pltpu.SemaphoreType.DMA,
        ],
    )
    def _gather_embeddings(
        user_emb, movie_emb, user, item, u_out, i_out,
        uix, iix, urows, irows, sem_u, sem_i,
    ):
        wid = lax.axis_index("s") * NC + lax.axis_index("c")
        base = wid * BPW
        pltpu.sync_copy(user.at[pl.ds(base, BPW)], uix)
        pltpu.sync_copy(item.at[pl.ds(base, BPW)], iix)
        cu = pltpu.async_copy(user_emb.at[uix], urows, sem_u)
        ci = pltpu.async_copy(movie_emb.at[iix], irows, sem_i)
        cu.wait()
        pltpu.sync_copy(urows, u_out.at[pl.ds(base, BPW)])
        ci.wait()
        pltpu.sync_copy(irows, i_out.at[pl.ds(base, BPW)])

    return _gather_embeddings


CH = 2048


def _mlp_body(u_ref, v_ref, w1a_ref, w1b_ref, b1_ref, w2_ref, b2_ref, o_ref):
    x = jnp.dot(u_ref[...], w1a_ref[...], preferred_element_type=jnp.float32)
    x = x + jnp.dot(v_ref[...], w1b_ref[...], preferred_element_type=jnp.float32)
    h = jnp.maximum(x + b1_ref[...], 0.0)
    o_ref[...] = jnp.dot(h, w2_ref[...], preferred_element_type=jnp.float32) + b2_ref[...]


def _mlp(u, v, w1a, w1b, b1, w2, b2):
    return pl.pallas_call(
        _mlp_body,
        grid=(B // CH,),
        in_specs=[
            pl.BlockSpec((CH, D), lambda i: (i, 0)),
            pl.BlockSpec((CH, D), lambda i: (i, 0)),
            pl.BlockSpec((D, H), lambda i: (0, 0)),
            pl.BlockSpec((D, H), lambda i: (0, 0)),
            pl.BlockSpec((1, H), lambda i: (0, 0)),
            pl.BlockSpec((H, 1), lambda i: (0, 0)),
            pl.BlockSpec((1, 1), lambda i: (0, 0)),
        ],
        out_specs=pl.BlockSpec((CH, 1), lambda i: (i, 0)),
        out_shape=jax.ShapeDtypeStruct((B, 1), jnp.float32),
    )(u, v, w1a, w1b, b1, w2, b2)


def kernel(user, item, user_emb, movie_emb, W1, b1, W2, b2):
    u_rows, i_rows = _gather_kernel()(
        user_emb, movie_emb, user.astype(jnp.int32), item.astype(jnp.int32)
    )
    out = _mlp(u_rows, i_rows, W1[:D], W1[D:], b1.reshape(1, H), W2, b2.reshape(1, 1))
    return out[:, 0]
